# baseline (speedup 1.0000x reference)
import functools

import jax
import jax.numpy as jnp
from jax import lax
from jax.experimental import pallas as pl
from jax.experimental.pallas import tpu as pltpu
from jax.experimental.pallas import tpu_sc as plsc

N = 100000
E = 1000000
D = 64
DW = 128

K = 16
RANGE = 6250
RSLICE = 392
RPAD = 16 * RSLICE
PADR = RPAD - RANGE
RINV = 1.0 / RANGE
ZB = 64

SEG = 2048
NV = SEG // 16
C = 32768
EPAD = 32 * C
NSEG = C // SEG
PADC = 128
CAP = 35072

BN = 896

_mesh = plsc.VectorSubcoreMesh(core_axis_name="c", subcore_axis_name="s")
_CP = pltpu.CompilerParams(needs_layout_passes=False)

def _popc(m):
    one = jnp.ones((16,), jnp.int32)
    zer = jnp.zeros((16,), jnp.int32)
    return jnp.sum(jnp.where(m, one, zer))


@functools.partial(
    pl.kernel,
    out_type=(
        jax.ShapeDtypeStruct((32 * CAP + 2048,), jnp.int32),
        jax.ShapeDtypeStruct((32 * CAP + 2048,), jnp.int32),
        jax.ShapeDtypeStruct((K * 32 * 16,), jnp.int32),
    ),
    mesh=_mesh,
    compiler_params=_CP,
    scratch_types=[
        pltpu.VMEM((SEG,), jnp.int32),
        pltpu.VMEM((SEG,), jnp.int32),
        pltpu.VMEM((CAP,), jnp.int32),
        pltpu.VMEM((CAP,), jnp.int32),
        pltpu.VMEM((16,), jnp.int32),
    ],
)
def _bin_kernel(ei_hbm, bsrc_hbm, bldst_hbm, cnts_hbm,
                dseg, sseg, bsrc_l, bldst_l, cvec):
    cc = lax.axis_index("c")
    ss = lax.axis_index("s")
    t = ss * 2 + cc
    base = t * C
    ii = lax.iota(jnp.int32, 16)
    one = jnp.ones((16,), jnp.int32)
    zer = jnp.zeros((16,), jnp.int32)

    def p1_seg(seg, cvs):
        pltpu.sync_copy(ei_hbm.at[pl.ds(EPAD + base + seg * SEG, SEG)], dseg)

        def p1_vec(i, cvs):
            d = dseg[pl.ds(i * 16, 16)]
            qd = ((d.astype(jnp.float32) + 0.5) * RINV).astype(jnp.int32)
            out = []
            for b in range(K):
                m = qd == b
                out.append(cvs[b] + jnp.where(m, one, zer))
            return tuple(out)

        return lax.fori_loop(0, NV, p1_vec, cvs)

    cvs = lax.fori_loop(0, NSEG, p1_seg, tuple(zer for _ in range(K)))
    cnts = [jnp.sum(cvs[b]) for b in range(K)]

    nch, off = [], []
    cur = jnp.int32(0)
    for b in range(K):
        n = (cnts[b] + (PADC - 1)) >> 7
        nch.append(n)
        off.append(cur)
        cur = cur + (n << 7)

    def p2_seg(seg, w):
        pltpu.sync_copy(ei_hbm.at[pl.ds(EPAD + base + seg * SEG, SEG)], dseg)
        pltpu.sync_copy(ei_hbm.at[pl.ds(base + seg * SEG, SEG)], sseg)

        def p2_vec(i, w):
            d = dseg[pl.ds(i * 16, 16)]
            sv = sseg[pl.ds(i * 16, 16)]
            qd = ((d.astype(jnp.float32) + 0.5) * RINV).astype(jnp.int32)
            qs = ((sv.astype(jnp.float32) + 0.5) * RINV).astype(jnp.int32)
            psv = sv + qs * PADR
            ld = d - qd * RANGE
            wo = []
            for b in range(K):
                m = qd == b
                plsc.store_compressed(bsrc_l.at[pl.ds(w[b], 16)], psv, mask=m)
                plsc.store_compressed(bldst_l.at[pl.ds(w[b], 16)], ld, mask=m)
                wo.append(w[b] + _popc(m))
            return tuple(wo)

        return lax.fori_loop(0, NV, p2_vec, w)

    w = lax.fori_loop(0, NSEG, p2_seg, tuple(off))

    dsrc = (ii * 997 + t * 7919) & 65535
    dldst = RANGE + ((ii + t) & 7)
    for b in range(K):
        endb = off[b] + (nch[b] << 7)

        def pad_body(j, wb):
            wv = wb + ii
            m = wv < endb
            plsc.store_scatter(bsrc_l, [wv], dsrc, mask=m)
            plsc.store_scatter(bldst_l, [wv], dldst, mask=m)
            return wb + 16

        lax.fori_loop(0, 8, pad_body, w[b])
        cvec[...] = jnp.where(ii == 0, off[b] >> 7, jnp.where(ii == 1, nch[b], 0))
        pltpu.sync_copy(cvec, cnts_hbm.at[pl.ds((b * 32 + t) * 16, 16)])

    pltpu.sync_copy(bsrc_l, bsrc_hbm.at[pl.ds(t * CAP, CAP)])
    pltpu.sync_copy(bldst_l, bldst_hbm.at[pl.ds(t * CAP, CAP)])


def _make_agg_kernel(compute_deg):
    out_types = [jax.ShapeDtypeStruct((K, RPAD, DW), jnp.float32)]
    if compute_deg:
        out_types.append(jax.ShapeDtypeStruct((K * RPAD,), jnp.float32))

    @functools.partial(
        pl.kernel,
        out_type=tuple(out_types),
        mesh=_mesh,
        compiler_params=_CP,
        scratch_types=[
            pltpu.VMEM((16, PADC), jnp.int32),
            pltpu.VMEM((16, PADC), jnp.int32),
            pltpu.VMEM((3, PADC, DW), jnp.float32),
            pltpu.VMEM((ZB, DW), jnp.float32),
            pltpu.VMEM((RSLICE,), jnp.float32),
            pltpu.VMEM((RSLICE,), jnp.float32),
            pltpu.VMEM((PADC,), jnp.float32),
            pltpu.VMEM((16,), jnp.int32),
            pltpu.VMEM_SHARED((RPAD, DW), jnp.float32),
            pltpu.VMEM_SHARED((RPAD,), jnp.float32),
            pltpu.SemaphoreType.DMA,
            pltpu.SemaphoreType.DMA,
            pltpu.SemaphoreType.DMA,
            pltpu.SemaphoreType.DMA,
            pltpu.SemaphoreType.DMA,
            pltpu.SemaphoreType.DMA,
            pltpu.SemaphoreType.DMA,
            pltpu.SemaphoreType.DMA,
            pltpu.SemaphoreType.DMA,
            pltpu.SemaphoreType.DMA,
        ],
    )
    def _agg(h_hbm, bsrc_hbm, bldst_hbm, cnts_hbm, agg_hbm, *rest):
        if compute_deg:
            deg_hbm = rest[0]
            rest = rest[1:]
        (sidx, didx, rows, zblk, zvec, degv, onesv, cvec, acc, deg_s,
         semi, semd,
         semg0, semg1, semg2, semg3,
         sems0, sems1, sems2, sems3) = rest
        semg = [semg0, semg1, semg2, semg3]
        sems = [sems0, sems1, sems2, sems3]
        cc = lax.axis_index("c")
        ss = lax.axis_index("s")
        ii = lax.iota(jnp.int32, 16)
        rowbase = ss * RSLICE

        zero16 = jnp.zeros((16,), jnp.float32)

        def zb(i, _):
            for kk in range(DW // 16):
                zblk[i, pl.ds(kk * 16, 16)] = zero16
            return 0

        lax.fori_loop(0, ZB, zb, 0)

        def zv(i, _):
            zvec[pl.ds(i * 16, 16)] = zero16
            return 0

        lax.fori_loop(0, RSLICE // 16, zv, 0)

        def ov(i, _):
            onesv[pl.ds(i * 16, 16)] = jnp.ones((16,), jnp.float32)
            return 0

        lax.fori_loop(0, PADC // 16, ov, 0)

        def one_pass(p, _):
            b = (K // 2) * cc + p

            zdescs = []
            for kz in range(RSLICE // ZB):
                zdescs.append(pltpu.make_async_copy(
                    zblk, acc.at[pl.ds(rowbase + kz * ZB, ZB), :], semi))
            rem = RSLICE % ZB
            if rem:
                zdescs.append(pltpu.make_async_copy(
                    zblk.at[pl.ds(0, rem), :],
                    acc.at[pl.ds(rowbase + (RSLICE // ZB) * ZB, rem), :],
                    semi))
            if compute_deg:
                zdescs.append(pltpu.make_async_copy(
                    zvec, deg_s.at[pl.ds(rowbase, RSLICE)], semi))
            for dsc in zdescs:
                dsc.start()
            for dsc in zdescs:
                dsc.wait()
            plsc.subcore_barrier()

            for qi in range(2):
                q = ss * 2 + qi
                pltpu.sync_copy(cnts_hbm.at[pl.ds((b * 32 + q) * 16, 16)], cvec)
                v = cvec[...]
                offc = jnp.max(jnp.where(ii == 0, v, 0))
                nch = jnp.max(jnp.where(ii == 1, v, 0))
                nblk = (nch + 15) >> 4

                def block(blk, _):
                    ch0 = (offc + blk * 16) * PADC

                    idescs = []
                    for jj in range(16):
                        st = ch0 + jj * PADC
                        idescs.append(pltpu.make_async_copy(
                            bsrc_hbm.at[pl.ds(q * CAP + st, PADC)], sidx.at[jj], semi))
                        idescs.append(pltpu.make_async_copy(
                            bldst_hbm.at[pl.ds(q * CAP + st, PADC)], didx.at[jj], semi))
                    for dsc in idescs:
                        dsc.start()
                    for dsc in idescs:
                        dsc.wait()

                    def gissue(jj):
                        @pl.when(blk * 16 + jj < nch)
                        def _():
                            pltpu.async_copy(
                                h_hbm.at[sidx.at[jj]], rows.at[jj % 3], semg[jj % 3])

                    def gwait(jj):
                        @pl.when(blk * 16 + jj < nch)
                        def _():
                            pltpu.make_async_copy(
                                h_hbm.at[sidx.at[jj]], rows.at[jj % 3], semg[jj % 3]
                            ).wait()

                    def sissue(jj):
                        @pl.when(blk * 16 + jj < nch)
                        def _():
                            pltpu.async_copy(
                                rows.at[jj % 3], acc.at[didx.at[jj]], sems[jj % 3],
                                add=True)
                            if compute_deg:
                                pltpu.async_copy(
                                    onesv, deg_s.at[didx.at[jj]], semd, add=True)

                    def swait(jj):
                        @pl.when(blk * 16 + jj < nch)
                        def _():
                            pltpu.make_async_copy(
                                rows.at[jj % 3], acc.at[didx.at[jj]], sems[jj % 3]
                            ).wait()
                            if compute_deg:
                                pltpu.make_async_copy(
                                    onesv, deg_s.at[didx.at[jj]], semd).wait()

                    for jj in range(2):
                        gissue(jj)
                    for jj in range(16):
                        gwait(jj)
                        sissue(jj)
                        if jj >= 1:
                            swait(jj - 1)
                        if jj + 2 < 16:
                            gissue(jj + 2)
                    swait(15)
                    return 0

                lax.fori_loop(0, nblk, block, 0)

            plsc.subcore_barrier()
            pltpu.sync_copy(
                acc.at[pl.ds(rowbase, RSLICE), :],
                agg_hbm.at[b, pl.ds(rowbase, RSLICE), :],
            )
            if compute_deg:
                pltpu.sync_copy(deg_s.at[pl.ds(rowbase, RSLICE)], degv)
                pltpu.sync_copy(degv, deg_hbm.at[pl.ds(b * RPAD + rowbase, RSLICE)])
            return 0

        lax.fori_loop(0, K // 2, one_pass, 0)

    return _agg


_agg_deg = _make_agg_kernel(True)
_agg_nodeg = _make_agg_kernel(False)


def _lin_body(h_ref, agg_ref, deg_ref, wt_ref, wb_ref, b_ref, o_ref, *, relu, odw):
    h = h_ref[0, :, :D]
    deg = jnp.maximum(deg_ref[0], 1.0)
    agg = agg_ref[0, :, :D] * (1.0 / deg)
    out = (
        jnp.dot(h, wt_ref[...], preferred_element_type=jnp.float32)
        + jnp.dot(agg, wb_ref[...], preferred_element_type=jnp.float32)
        + b_ref[...]
    )
    if relu:
        out = jnp.maximum(out, 0.0)
    if odw:
        out = jnp.concatenate([out, jnp.zeros((BN, DW - D), jnp.float32)], axis=1)
    o_ref[0, :, :] = out


def _linear(h_p, agg_p, deg_p, W, b, relu, odw):
    wt = W[:D]
    wb = W[D:]
    b2 = b.reshape(1, D)
    deg3 = deg_p.reshape(K, RPAD, 1)
    nb = RPAD // BN
    ow = DW if odw else D
    return pl.pallas_call(
        functools.partial(_lin_body, relu=relu, odw=odw),
        grid=(K, nb),
        in_specs=[
            pl.BlockSpec((1, BN, DW), lambda i, j: (i, j, 0)),
            pl.BlockSpec((1, BN, DW), lambda i, j: (i, j, 0)),
            pl.BlockSpec((1, BN, 1), lambda i, j: (i, j, 0)),
            pl.BlockSpec((D, D), lambda i, j: (0, 0)),
            pl.BlockSpec((D, D), lambda i, j: (0, 0)),
            pl.BlockSpec((1, D), lambda i, j: (0, 0)),
        ],
        out_specs=pl.BlockSpec((1, BN, ow), lambda i, j: (i, j, 0)),
        out_shape=jax.ShapeDtypeStruct((K, RPAD, ow), jnp.float32),
    )(h_p, agg_p, deg3, wt, wb, b2)


def kernel(x, edge_index, W0, b0, W1, b1):
    ei_p = jnp.pad(edge_index, ((0, 0), (0, EPAD - E)), constant_values=N)

    bsrc, bldst, cnts = _bin_kernel(ei_p.reshape(2 * EPAD))

    x_p = jnp.pad(x.reshape(K, RANGE, D), ((0, 0), (0, PADR), (0, DW - D)))
    agg0, deg_p = _agg_deg(x_p.reshape(K * RPAD, DW), bsrc, bldst, cnts)
    h1_p = _linear(x_p, agg0, deg_p, W0, b0, relu=True, odw=True)

    agg1 = _agg_nodeg(h1_p.reshape(K * RPAD, DW), bsrc, bldst, cnts)
    if isinstance(agg1, (tuple, list)):
        agg1 = agg1[0]
    h2_p = _linear(h1_p, agg1, deg_p, W1, b1, relu=False, odw=False)
    return h2_p[:, :RANGE, :].reshape(N, D)
